# baseline (speedup 1.0000x reference)
import functools

import jax
import jax.numpy as jnp
from jax import lax
from jax.experimental import pallas as pl
from jax.experimental.pallas import tpu as pltpu
from jax.experimental.pallas import tpu_sc as plsc

VOCAB = 1000000
EMBED = 16
BATCH = 16384
LANES = 16
NW = 32
BPW = BATCH // NW
CHUNK = 128
NCHUNK = BPW // CHUNK
GPW = BPW // LANES


def _sc_call(idx, table, wb, bvec):
    mesh = plsc.VectorSubcoreMesh(core_axis_name="c", subcore_axis_name="s")

    @functools.partial(
        pl.kernel,
        mesh=mesh,
        compiler_params=pltpu.CompilerParams(
            needs_layout_passes=False, use_tc_tiling_on_sc=False
        ),
        out_type=jax.ShapeDtypeStruct((NW, BPW), jnp.float32),
        scratch_types=[
            pltpu.VMEM((BPW,), jnp.int32),
            pltpu.VMEM((BPW, EMBED), jnp.float32),
            pltpu.VMEM((EMBED, LANES), jnp.float32),
            pltpu.VMEM((LANES,), jnp.float32),
            pltpu.VMEM((BPW,), jnp.float32),
            pltpu.SemaphoreType.DMA,
        ],
    )
    def sc_kernel(idx_hbm, table_hbm, wb_hbm, b_hbm, out_hbm,
                  idx_v, rows_v, wb_v, b_v, out_v, sem):
        wid = lax.axis_index("s") * 2 + lax.axis_index("c")
        pltpu.sync_copy(idx_hbm.at[wid], idx_v)
        pltpu.sync_copy(wb_hbm, wb_v)
        pltpu.sync_copy(b_hbm, b_v)

        copies = [
            pltpu.async_copy(
                table_hbm.at[idx_v.at[pl.ds(c * CHUNK, CHUNK)]],
                rows_v.at[pl.ds(c * CHUNK, CHUNK)],
                sem,
            )
            for c in range(NCHUNK)
        ]

        wrows = [wb_v[j, :] for j in range(EMBED)]
        bias = b_v[...]
        base_iota = lax.iota(jnp.int32, LANES)

        gpc = CHUNK // LANES
        for c in range(NCHUNK):
            copies[c].wait()
            for g in range(gpc):
                off = c * CHUNK + g * LANES
                i_ids = base_iota + off
                acc = bias
                for j in range(EMBED):
                    col = plsc.load_gather(
                        rows_v,
                        [i_ids, jnp.full((LANES,), j, jnp.int32)],
                    )
                    acc = acc + col * wrows[j]
                out_v[pl.ds(off, LANES)] = acc

        pltpu.sync_copy(out_v, out_hbm.at[wid])

    return sc_kernel(idx, table, wb, bvec)


def kernel(x, table, W, b):
    idx = x.reshape(NW, BPW).astype(jnp.int32)
    wb = jnp.broadcast_to(
        W.astype(jnp.float32).reshape(EMBED, 1), (EMBED, LANES)
    )
    bvec = jnp.broadcast_to(b.astype(jnp.float32).reshape(1), (LANES,))
    out = _sc_call(idx, table.astype(jnp.float32), wb, bvec)
    return out.reshape(BATCH, 1)
